# baseline (speedup 1.0000x reference)
import jax
import jax.numpy as jnp
from jax.experimental import pallas as pl
from jax.experimental.pallas import tpu as pltpu

_CLS_W = 1.0
_BBOX_W = 10.0
_BS = 8
_A = 9
_HW = 2500
_N = _A * _HW


def _loss_body(logits_ref, labels_a_ref, labels_p_ref, bbox_ref, gt_ref,
               out_ref):
    lg = logits_ref[...]
    tg = labels_a_ref[...].astype(jnp.float32)
    bce_sum = jnp.sum(
        jnp.maximum(lg, 0.0) - lg * tg + jnp.log1p(jnp.exp(-jnp.abs(lg))))
    npos = jnp.sum(tg)

    mp = labels_p_ref[...].astype(jnp.float32)
    a_i = jax.lax.broadcasted_iota(jnp.int32, (_A, 4 * _A), 0)
    ch_i = jax.lax.broadcasted_iota(jnp.int32, (_A, 4 * _A), 1)
    pmat = (a_i == ch_i // 4).astype(jnp.float32)
    mask36 = jax.lax.dot_general(
        mp, pmat, dimension_numbers=(((2,), (0,)), ((), ())),
        preferred_element_type=jnp.float32)

    bt = jnp.transpose(bbox_ref[...], (0, 2, 1))
    diff = bt - gt_ref[...]
    ad = jnp.abs(diff)
    sl1 = jnp.where(ad < 1.0, 0.5 * diff * diff, ad - 0.5)
    masked_sum = jnp.sum(sl1 * mask36)

    cls_loss = bce_sum / (_BS * _N)
    denom = jnp.maximum(2.0 * npos, 1.0)
    bbox_loss = jnp.where(npos > 0.0, masked_sum / denom, 0.0)
    out_ref[0, 0] = _CLS_W * cls_loss + _BBOX_W * bbox_loss


def kernel(rpn_cls_logits, rpn_bbox_reg, anchor_labels, anchor_gt_boxes):
    logits = rpn_cls_logits.reshape(_BS * _A, _HW)
    labels_a = anchor_labels.reshape(_BS * _A, _HW)
    labels_p = anchor_labels.reshape(_BS, _HW, _A)
    bbox = rpn_bbox_reg.reshape(_BS, 4 * _A, _HW)
    gt = anchor_gt_boxes.reshape(_BS, _HW, 4 * _A)

    out = pl.pallas_call(
        _loss_body,
        out_shape=jax.ShapeDtypeStruct((1, 1), jnp.float32),
        out_specs=pl.BlockSpec(memory_space=pltpu.SMEM),
    )(logits, labels_a, labels_p, bbox, gt)
    return out[0, 0]
